# baseline (speedup 1.0000x reference)
import functools

import jax
import jax.numpy as jnp
from jax.experimental import pallas as pl
from jax.experimental.pallas import tpu as pltpu

NUM_ITEMS = 8192
DIM = 10000
BLOCK_R = 256
EPS = 1e-8


def _body(noisy_ref, vec_ref, clean_ref, idx_ref, sim_ref,
          best_sim_s, best_idx_s):
    i = pl.program_id(0)

    @pl.when(i == 0)
    def _init():
        best_sim_s[0] = -jnp.inf
        best_idx_s[0] = 0

    x = vec_ref[...]
    n = noisy_ref[...]
    dot = jnp.sum(x * n, axis=1, keepdims=True)
    sq = jnp.sum(x * x, axis=1, keepdims=True)
    nn = jnp.maximum(jnp.sqrt(jnp.sum(n * n)), EPS)
    sims = dot / (jnp.maximum(jnp.sqrt(sq), EPS) * nn)

    m = jnp.max(sims)
    rows = jax.lax.broadcasted_iota(jnp.int32, (BLOCK_R, 1), 0)
    bi = jnp.min(jnp.where(sims == m, rows, NUM_ITEMS))

    @pl.when(m > best_sim_s[0])
    def _update():
        best_sim_s[0] = m
        best_idx_s[0] = i * BLOCK_R + bi
        clean_ref[...] = vec_ref[pl.ds(bi, 1), :]

    @pl.when(i == pl.num_programs(0) - 1)
    def _finalize():
        idx_ref[0, 0] = best_idx_s[0]
        sim_ref[0, 0] = best_sim_s[0]


@jax.jit
def kernel(noisy, vectors):
    noisy2d = noisy.reshape(1, DIM)
    grid = (NUM_ITEMS // BLOCK_R,)
    clean, idx, sim = pl.pallas_call(
        _body,
        grid=grid,
        in_specs=[
            pl.BlockSpec((1, DIM), lambda i: (0, 0)),
            pl.BlockSpec((BLOCK_R, DIM), lambda i: (i, 0)),
        ],
        out_specs=[
            pl.BlockSpec((1, DIM), lambda i: (0, 0)),
            pl.BlockSpec(memory_space=pltpu.SMEM),
            pl.BlockSpec(memory_space=pltpu.SMEM),
        ],
        out_shape=[
            jax.ShapeDtypeStruct((1, DIM), jnp.float32),
            jax.ShapeDtypeStruct((1, 1), jnp.int32),
            jax.ShapeDtypeStruct((1, 1), jnp.float32),
        ],
        scratch_shapes=[
            pltpu.SMEM((1,), jnp.float32),
            pltpu.SMEM((1,), jnp.int32),
        ],
    )(noisy2d, vectors)
    return clean[0], idx[0, 0], sim[0, 0]
